# baseline (speedup 1.0000x reference)
import functools

import jax
import jax.numpy as jnp
from jax import lax
from jax.experimental import pallas as pl
from jax.experimental.pallas import tpu as pltpu
from jax.experimental.pallas import tpu_sc as plsc

BATCH = 1024
HIST = 200
FEAT = 128
VOCAB = 1000

NC, NS, L = 2, 16, 16
NW = NC * NS
TOT = BATCH * HIST
BPW = TOT // NW
IDX_CH = 128
CH = 128
NCH = BPW // CH
NBUF = 6
LAG = 3
SP_SLOT = (True, False, True, False, True, False)

_mesh = plsc.VectorSubcoreMesh(
    core_axis_name="c", subcore_axis_name="s", num_cores=NC, num_subcores=NS
)


@functools.partial(
    pl.kernel,
    out_type=jax.ShapeDtypeStruct((TOT, FEAT), jnp.float32),
    mesh=_mesh,
    scratch_types=[
        pltpu.VMEM((BPW,), jnp.int32),
        pltpu.VMEM((NBUF, CH, FEAT), jnp.float32),
        pltpu.VMEM_SHARED((VOCAB, FEAT), jnp.float32),
        [pltpu.SemaphoreType.DMA] * NBUF,
        [pltpu.SemaphoreType.DMA] * NBUF,
    ],
)
def _span_gather(idx_hbm, table_hbm, out_hbm, idx_v, bufs, table_sp, gsem, ssem):
    wid = lax.axis_index("s") * NC + lax.axis_index("c")
    sid = lax.axis_index("s")
    base = wid * BPW

    @pl.when(sid == 0)
    def _stage():
        pltpu.sync_copy(table_hbm, table_sp)

    plsc.subcore_barrier()

    def sub1_chunk(c):
        @pl.loop(0, CH // L)
        def _(j):
            sl = pl.ds(c * CH + j * L, L)
            idx_v[sl] = idx_v[sl] - 1

    def issue_gather(c, slot):
        table = table_sp if SP_SLOT[slot] else table_hbm
        buf = bufs.at[slot]
        for h in range(CH // IDX_CH):
            idx_slice = idx_v.at[pl.ds(c * CH + h * IDX_CH, IDX_CH)]
            pltpu.async_copy(
                table.at[idx_slice], buf.at[pl.ds(h * IDX_CH, IDX_CH)], gsem[slot]
            )

    def wait_gather(slot):
        pltpu.make_async_copy(
            table_hbm.at[pl.ds(0, CH)], bufs.at[slot], gsem[slot]
        ).wait()

    def issue_scatter(c, slot):
        pltpu.async_copy(bufs.at[slot], out_hbm.at[pl.ds(base + c * CH, CH)], ssem[slot])

    def wait_scatter(slot):
        pltpu.make_async_copy(
            bufs.at[slot], out_hbm.at[pl.ds(base, CH)], ssem[slot]
        ).wait()

    pltpu.sync_copy(idx_hbm.at[wid], idx_v)

    for b in range(NBUF):
        sub1_chunk(b)
    for b in range(NBUF):
        issue_gather(b, b)
        if b >= LAG:
            wait_gather(b - LAG)
            issue_scatter(b - LAG, b - LAG)

    ROUNDS = (NCH - 2 * NBUF - 1) // NBUF
    @pl.loop(0, ROUNDS)
    def _round(i):
        for b in range(NBUF):
            c = NBUF + i * NBUF + b
            sub1_chunk(c)
            wait_scatter(b)
            issue_gather(c, b)
            pb = (b - LAG) % NBUF
            wait_gather(pb)
            issue_scatter(c - LAG, pb)

    for c in range(NBUF + ROUNDS * NBUF, NCH):
        b = c % NBUF
        sub1_chunk(c)
        wait_scatter(b)
        issue_gather(c, b)
        pb = (c - LAG) % NBUF
        wait_gather(pb)
        issue_scatter(c - LAG, pb)

    for c in range(NCH - LAG, NCH):
        slot = c % NBUF
        wait_gather(slot)
        issue_scatter(c, slot)
    for c in range(NCH - NBUF, NCH):
        wait_scatter(c % NBUF)


def kernel(span_width, span_width_embeddings):
    idx = span_width.reshape(NW, BPW)
    out = _span_gather(idx, span_width_embeddings)
    return out.reshape(BATCH, HIST, FEAT)


# speedup vs baseline: 1.6483x; 1.6483x over previous
import functools

import jax
import jax.numpy as jnp
from jax import lax
from jax.experimental import pallas as pl
from jax.experimental.pallas import tpu as pltpu
from jax.experimental.pallas import tpu_sc as plsc

BATCH = 1024
HIST = 200
FEAT = 128
VOCAB = 1000

NC, NS, L = 2, 16, 16
NW = NC * NS
TOT = BATCH * HIST
BPW = TOT // NW
CH = 128
NCH = BPW // CH
NBUF = 5
LAG = 2
VSP = VOCAB + 8

_mesh = plsc.VectorSubcoreMesh(
    core_axis_name="c", subcore_axis_name="s", num_cores=NC, num_subcores=NS
)


@functools.partial(
    pl.kernel,
    out_type=jax.ShapeDtypeStruct((TOT, FEAT), jnp.float32),
    mesh=_mesh,
    scratch_types=[
        pltpu.VMEM((BPW,), jnp.int32),
        pltpu.VMEM((NBUF, CH, FEAT), jnp.float32),
        pltpu.VMEM_SHARED((VSP, FEAT), jnp.float32),
        [pltpu.SemaphoreType.DMA] * NBUF,
        [pltpu.SemaphoreType.DMA] * NBUF,
    ],
)
def _span_gather(idx_hbm, table_hbm, out_hbm, idx_v, bufs, table_sp, gsem, ssem):
    wid = lax.axis_index("s") * NC + lax.axis_index("c")
    sid = lax.axis_index("s")
    base = wid * BPW

    @pl.when(sid == 0)
    def _stage():
        pltpu.sync_copy(table_hbm, table_sp.at[pl.ds(1, VOCAB)])

    plsc.subcore_barrier()

    def issue_gather(c, slot):
        idx_slice = idx_v.at[pl.ds(c * CH, CH)]
        pltpu.async_copy(table_sp.at[idx_slice], bufs.at[slot], gsem[slot])

    def wait_gather(slot):
        pltpu.make_async_copy(
            table_hbm.at[pl.ds(0, CH)], bufs.at[slot], gsem[slot]
        ).wait()

    def issue_scatter(c, slot):
        pltpu.async_copy(bufs.at[slot], out_hbm.at[pl.ds(base + c * CH, CH)], ssem[slot])

    def wait_scatter(slot):
        pltpu.make_async_copy(
            bufs.at[slot], out_hbm.at[pl.ds(base, CH)], ssem[slot]
        ).wait()

    pltpu.sync_copy(idx_hbm.at[wid], idx_v)

    for b in range(NBUF):
        issue_gather(b, b)
        if b >= LAG:
            wait_gather(b - LAG)
            issue_scatter(b - LAG, b - LAG)

    ROUNDS = (NCH - 2 * NBUF) // NBUF
    @pl.loop(0, ROUNDS)
    def _round(i):
        for b in range(NBUF):
            c = NBUF + i * NBUF + b
            wait_scatter(b)
            issue_gather(c, b)
            pb = (b - LAG) % NBUF
            wait_gather(pb)
            issue_scatter(c - LAG, pb)

    for c in range(NBUF + ROUNDS * NBUF, NCH):
        b = c % NBUF
        wait_scatter(b)
        issue_gather(c, b)
        pb = (c - LAG) % NBUF
        wait_gather(pb)
        issue_scatter(c - LAG, pb)

    for c in range(NCH - LAG, NCH):
        slot = c % NBUF
        wait_gather(slot)
        issue_scatter(c, slot)
    for c in range(NCH - NBUF, NCH):
        wait_scatter(c % NBUF)


def kernel(span_width, span_width_embeddings):
    idx = span_width.reshape(NW, BPW)
    out = _span_gather(idx, span_width_embeddings)
    return out.reshape(BATCH, HIST, FEAT)
